# Initial kernel scaffold; baseline (speedup 1.0000x reference)
import jax
import jax.numpy as jnp
from jax.experimental import pallas as pl


def kernel(inputs, Wg, bg, We, be):
    raise NotImplementedError("write your pallas kernel here")


# speedup vs baseline: 3.3787x; 3.3787x over previous
import functools

import jax
import jax.numpy as jnp
from jax import lax
from jax.experimental import pallas as pl
from jax.experimental.pallas import tpu as pltpu
from jax.experimental.pallas import tpu_sc as plsc

N = 8192
D = 768
DO = 768
E = 64
CAP = 320

NC = 2
NS = 16
NW = NC * NS

TB = 128
NB = N // TB
TRASH = E * CAP
XE_ROWS = (E + 1) * CAP

TW = N // NW
CH = 64
NCH = TW // CH


def _gating_body(x_ref, wg_ref, bg_ref,
                 scat1_ref, scat2_ref, comb1_ref, comb2_ref, g1_ref, g2_ref,
                 carry_ref):
    b = pl.program_id(0)

    @pl.when(b == 0)
    def _():
        carry_ref[...] = jnp.zeros_like(carry_ref)

    logits = jnp.dot(x_ref[...].astype(jnp.bfloat16),
                     wg_ref[...].astype(jnp.bfloat16),
                     preferred_element_type=jnp.float32) + bg_ref[...]

    iota_e = lax.broadcasted_iota(jnp.int32, (TB, E), 1)
    m1 = jnp.max(logits, axis=1, keepdims=True)
    a1 = jnp.min(jnp.where(logits == m1, iota_e, E), axis=1, keepdims=True)
    oh1 = iota_e == a1
    masked = jnp.where(oh1, -jnp.inf, logits)
    m2 = jnp.max(masked, axis=1, keepdims=True)
    a2 = jnp.min(jnp.where(masked == m2, iota_e, E), axis=1, keepdims=True)
    oh2 = iota_e == a2

    t = jnp.exp(m2 - m1)
    den = 1.0 + t
    g1 = 1.0 / den
    g2 = t / den

    ohsum = oh1.astype(jnp.float32) + oh2.astype(jnp.float32)
    ii = lax.broadcasted_iota(jnp.int32, (TB, TB), 0)
    jj = lax.broadcasted_iota(jnp.int32, (TB, TB), 1)
    tri = (jj < ii).astype(jnp.float32)
    S = jnp.dot(tri, ohsum, preferred_element_type=jnp.float32) + carry_ref[...]
    carry_ref[...] = carry_ref[...] + jnp.sum(ohsum, axis=0, keepdims=True)

    pos1 = jnp.sum(jnp.where(oh1, S, 0.0), axis=1, keepdims=True).astype(jnp.int32)
    pos2 = jnp.sum(jnp.where(oh2, S, 0.0), axis=1, keepdims=True).astype(jnp.int32)
    v1 = pos1 < CAP
    v2 = pos2 < CAP
    base1 = a1 * CAP
    base2 = a2 * CAP
    scat1_ref[...] = jnp.where(v1, base1 + pos1, TRASH)
    scat2_ref[...] = jnp.where(v2, base2 + pos2, TRASH)
    comb1_ref[...] = jnp.where(v1, base1 + pos1, base1)
    comb2_ref[...] = jnp.where(v2, base2 + pos2, base2)
    g1_ref[...] = jnp.where(v1, g1, 0.0)
    g2_ref[...] = jnp.where(v2, g2, 0.0)


def _gating(x, wg, bg2):
    col_i = jax.ShapeDtypeStruct((N, 1), jnp.int32)
    col_f = jax.ShapeDtypeStruct((N, 1), jnp.float32)
    colspec = pl.BlockSpec((TB, 1), lambda b: (b, 0))
    return pl.pallas_call(
        _gating_body,
        grid=(NB,),
        in_specs=[
            pl.BlockSpec((TB, D), lambda b: (b, 0)),
            pl.BlockSpec((D, E), lambda b: (0, 0)),
            pl.BlockSpec((1, E), lambda b: (0, 0)),
        ],
        out_specs=[colspec] * 6,
        out_shape=[col_i, col_i, col_i, col_i, col_f, col_f],
        scratch_shapes=[pltpu.VMEM((1, E), jnp.float32)],
    )(x, wg, bg2)


def _sc_scatter_body(x_hbm, s1_hbm, s2_hbm, xe_hbm, xv, i1, i2, sem):
    w = lax.axis_index("s") * NC + lax.axis_index("c")
    for j in range(NCH):
        base = w * TW + j * CH
        pltpu.sync_copy(x_hbm.at[pl.ds(base, CH)], xv)
        pltpu.sync_copy(s1_hbm.at[pl.ds(base, CH)], i1)
        pltpu.sync_copy(s2_hbm.at[pl.ds(base, CH)], i2)
        c1 = pltpu.async_copy(xv, xe_hbm.at[i1], sem)
        c2 = pltpu.async_copy(xv, xe_hbm.at[i2], sem)
        c1.wait()
        c2.wait()


_SC_MESH = dict(core_axis_name="c", subcore_axis_name="s",
                num_cores=NC, num_subcores=NS)


def _make_sc_scatter():
    return pl.kernel(
        _sc_scatter_body,
        out_type=jax.ShapeDtypeStruct((XE_ROWS, D), jnp.float32),
        mesh=plsc.VectorSubcoreMesh(**_SC_MESH),
        scratch_types=[
            pltpu.VMEM((CH, D), jnp.float32),
            pltpu.VMEM((CH,), jnp.int32),
            pltpu.VMEM((CH,), jnp.int32),
            pltpu.SemaphoreType.DMA,
        ],
    )


def _make_sc_gather():
    return pl.kernel(
        _sc_gather_body,
        out_type=(jax.ShapeDtypeStruct((N, DO), jnp.float32),
                  jax.ShapeDtypeStruct((N, DO), jnp.float32)),
        mesh=plsc.VectorSubcoreMesh(**_SC_MESH),
        scratch_types=[
            pltpu.VMEM((CH, DO), jnp.float32),
            pltpu.VMEM((CH, DO), jnp.float32),
            pltpu.VMEM((CH,), jnp.int32),
            pltpu.VMEM((CH,), jnp.int32),
            pltpu.SemaphoreType.DMA,
        ],
    )


def _ffn_body(xe_ref, we_ref, be_ref, y_ref):
    xb = xe_ref[...].astype(jnp.bfloat16)
    wb = we_ref[0].astype(jnp.bfloat16)
    y_ref[...] = (jnp.dot(xb, wb, preferred_element_type=jnp.float32)
                  + be_ref[0])


def _ffn(xe, we, be):
    return pl.pallas_call(
        _ffn_body,
        grid=(E,),
        in_specs=[
            pl.BlockSpec((CAP, D), lambda e: (e, 0)),
            pl.BlockSpec((1, D, DO), lambda e: (e, 0, 0)),
            pl.BlockSpec((1, 1, DO), lambda e: (e, 0, 0)),
        ],
        out_specs=pl.BlockSpec((CAP, DO), lambda e: (e, 0)),
        out_shape=jax.ShapeDtypeStruct((E * CAP, DO), jnp.float32),
    )(xe, we, be.reshape(E, 1, DO))


def _sc_gather_body(y_hbm, c1_hbm, c2_hbm, y1_hbm, y2_hbm, yv1, yv2, i1, i2, sem):
    w = lax.axis_index("s") * NC + lax.axis_index("c")
    for j in range(NCH):
        base = w * TW + j * CH
        pltpu.sync_copy(c1_hbm.at[pl.ds(base, CH)], i1)
        pltpu.sync_copy(c2_hbm.at[pl.ds(base, CH)], i2)
        d1 = pltpu.async_copy(y_hbm.at[i1], yv1, sem)
        d2 = pltpu.async_copy(y_hbm.at[i2], yv2, sem)
        d1.wait()
        d2.wait()
        pltpu.sync_copy(yv1, y1_hbm.at[pl.ds(base, CH)])
        pltpu.sync_copy(yv2, y2_hbm.at[pl.ds(base, CH)])


CB = 512


def _combine_body(g1_ref, g2_ref, y1_ref, y2_ref, o_ref):
    o_ref[...] = g1_ref[...] * y1_ref[...] + g2_ref[...] * y2_ref[...]


def _combine(g1, g2, y1, y2):
    rowspec = pl.BlockSpec((CB, DO), lambda b: (b, 0))
    colspec = pl.BlockSpec((CB, 1), lambda b: (b, 0))
    return pl.pallas_call(
        _combine_body,
        grid=(N // CB,),
        in_specs=[colspec, colspec, rowspec, rowspec],
        out_specs=rowspec,
        out_shape=jax.ShapeDtypeStruct((N, DO), jnp.float32),
    )(g1, g2, y1, y2)


def kernel(inputs, Wg, bg, We, be):
    bg2 = bg.reshape(1, E)
    scat1, scat2, comb1, comb2, g1, g2 = _gating(inputs, Wg, bg2)
    xe = _make_sc_scatter()(inputs, scat1.reshape(N), scat2.reshape(N))
    y = _ffn(xe, We, be)
    y1, y2 = _make_sc_gather()(y, comb1.reshape(N), comb2.reshape(N))
    return _combine(g1, g2, y1, y2)
